# baseline (speedup 1.0000x reference)
import functools

import jax
import jax.numpy as jnp
from jax import lax
from jax.experimental import pallas as pl
from jax.experimental.pallas import tpu as pltpu
from jax.experimental.pallas import tpu_sc as plsc

F32 = jnp.float32


def _edge_proj_body(ea_ref, we_ref, be_ref, out_ref):
  out_ref[...] = (
      jnp.dot(ea_ref[...], we_ref[...], preferred_element_type=F32)
      + be_ref[...]
  )


def _edge_proj(edge_attr, We_l, be_l):
  E, K = edge_attr.shape
  H = We_l.shape[1]
  BE = 4096
  grid = E // BE
  return pl.pallas_call(
      _edge_proj_body,
      grid=(grid,),
      in_specs=[
          pl.BlockSpec((BE, K), lambda i: (i, 0)),
          pl.BlockSpec((K, H), lambda i: (0, 0)),
          pl.BlockSpec((1, H), lambda i: (0, 0)),
      ],
      out_specs=pl.BlockSpec((BE, H), lambda i: (i, 0)),
      out_shape=jax.ShapeDtypeStruct((E, H), F32),
  )(edge_attr, We_l, be_l.reshape(1, H))


def _node_update_body(relu_out, h_ref, a0_ref, a1_ref, w1_ref, b1_ref,
                      w2_ref, b2_ref, out_ref):
  z = h_ref[...] + a0_ref[...] + a1_ref[...]
  z = jnp.maximum(
      jnp.dot(z, w1_ref[...], preferred_element_type=F32) + b1_ref[...], 0.0)
  z = jnp.dot(z, w2_ref[...], preferred_element_type=F32) + b2_ref[...]
  if relu_out:
    z = jnp.maximum(z, 0.0)
  out_ref[...] = z


def _node_update(h, a0, a1, W1_l, b1_l, W2_l, b2_l, relu_out):
  N, H = h.shape
  BN = 2000
  grid = N // BN
  row_spec = pl.BlockSpec((BN, H), lambda i: (i, 0))
  mat_spec = pl.BlockSpec((H, H), lambda i: (0, 0))
  vec_spec = pl.BlockSpec((1, H), lambda i: (0, 0))
  return pl.pallas_call(
      functools.partial(_node_update_body, relu_out),
      grid=(grid,),
      in_specs=[row_spec, row_spec, row_spec, mat_spec, vec_spec,
                mat_spec, vec_spec],
      out_specs=row_spec,
      out_shape=jax.ShapeDtypeStruct((N, H), F32),
  )(h, a0, a1, W1_l, b1_l.reshape(1, H), W2_l, b2_l.reshape(1, H))


_NC, _NS = 2, 16
_NW = _NC * _NS
_CHUNK = 64


def _make_sc_edge_pass(N, E_pad, H, NCH):
  EW = NCH * _CHUNK
  assert EW * _NW == E_pad and NCH % 2 == 0
  rows_per_tile = ((N + _NS - 1) // _NS + 7) // 8 * 8
  N_pad = rows_per_tile * _NS
  assert N_pad >= N + 1
  NP2 = NCH // 2

  def body(h_hbm, e_hbm, src_hbm, dst_hbm, zeros_hbm, out_hbm,
           acc_sh, rows0, rows1, ev0, ev1,
           srcl0, srcl1, dstl0, dstl1, dsts0, dsts1,
           sem_i0, sem_i1, sem_g0, sem_g1, sem_e0, sem_e1, sem_s0, sem_s1):
    c = lax.axis_index("c")
    s = lax.axis_index("s")
    wid = c * _NS + s
    ebase = wid * EW

    rows = (rows0, rows1)
    ev = (ev0, ev1)
    srcl = (srcl0, srcl1)
    dstl = (dstl0, dstl1)
    dsts = (dsts0, dsts1)
    sem_i = (sem_i0, sem_i1)
    sem_g = (sem_g0, sem_g1)
    sem_e = (sem_e0, sem_e1)
    sem_s = (sem_s0, sem_s1)

    pltpu.sync_copy(zeros_hbm.at[pl.ds(s * rows_per_tile, rows_per_tile)],
                    acc_sh.at[pl.ds(s * rows_per_tile, rows_per_tile)])
    plsc.subcore_barrier()

    def issue_idx(j, p):
      base = ebase + j * _CHUNK
      pltpu.async_copy(src_hbm.at[pl.ds(base, _CHUNK)], srcl[p], sem_i[p])
      pltpu.async_copy(dst_hbm.at[pl.ds(base, _CHUNK)], dstl[p], sem_i[p])

    def wait_idx(p):
      pltpu.make_async_copy(src_hbm.at[pl.ds(0, _CHUNK)], srcl[p],
                            sem_i[p]).wait()
      pltpu.make_async_copy(src_hbm.at[pl.ds(0, _CHUNK)], dstl[p],
                            sem_i[p]).wait()

    def issue_inputs(j, p):
      pltpu.async_copy(h_hbm.at[srcl[p]], rows[p], sem_g[p])
      pltpu.async_copy(e_hbm.at[pl.ds(ebase + j * _CHUNK, _CHUNK)],
                       ev[p], sem_e[p])

    def wait_inputs(p):
      pltpu.make_async_copy(e_hbm.at[pl.ds(0, _CHUNK)], rows[p],
                            sem_g[p]).wait()
      pltpu.make_async_copy(e_hbm.at[pl.ds(0, _CHUNK)], ev[p],
                            sem_e[p]).wait()

    def compute(p):
      rp, ep = rows[p], ev[p]

      def row(b, carry):
        for jj in range(H // 16):
          sl = pl.ds(jj * 16, 16)
          rp[b, sl] = jnp.maximum(rp[b, sl] + ep[b, sl], 0.0)
        return carry
      lax.fori_loop(0, _CHUNK, row, 0)

    def save_dst(p):
      for g in range(_CHUNK // 16):
        sl = pl.ds(g * 16, 16)
        dsts[p][sl] = dstl[p][sl]

    def issue_scatter(p):
      pltpu.async_copy(rows[p], acc_sh.at[dsts[p]], sem_s[p], add=True)

    def wait_scatter(p):
      pltpu.make_async_copy(rows[p], acc_sh.at[dsts[p]], sem_s[p]).wait()

    def step(j0, it, p):
      q = 1 - p
      j = j0 + p
      wait_inputs(p)
      save_dst(p)

      def _idx2():
        issue_idx(j + 2, p)
      pl.when(it < NP2 - 1)(_idx2)
      compute(p)
      if p == 0:
        pl.when(it > 0)(lambda: wait_scatter(q))
      else:
        wait_scatter(q)

      def _next_inputs():
        wait_idx(q)
        issue_inputs(j + 1, q)
      if p == 0:
        _next_inputs()
      else:
        pl.when(it < NP2 - 1)(_next_inputs)
      issue_scatter(p)

    issue_idx(0, 0)
    issue_idx(1, 1)
    wait_idx(0)
    issue_inputs(0, 0)

    def pair(it, carry):
      j0 = 2 * it
      step(j0, it, 0)
      step(j0, it, 1)
      return carry
    lax.fori_loop(0, NP2, pair, 0)
    wait_scatter(1)

    plsc.subcore_barrier()
    pltpu.sync_copy(acc_sh.at[pl.ds(s * rows_per_tile, rows_per_tile)],
                    out_hbm.at[c, pl.ds(s * rows_per_tile, rows_per_tile)])

  mesh = plsc.VectorSubcoreMesh(core_axis_name="c", subcore_axis_name="s")
  scratch = [
      pltpu.VMEM_SHARED((N_pad, H), F32),
      pltpu.VMEM((_CHUNK, H), F32),
      pltpu.VMEM((_CHUNK, H), F32),
      pltpu.VMEM((_CHUNK, H), F32),
      pltpu.VMEM((_CHUNK, H), F32),
      pltpu.VMEM((_CHUNK,), jnp.int32),
      pltpu.VMEM((_CHUNK,), jnp.int32),
      pltpu.VMEM((_CHUNK,), jnp.int32),
      pltpu.VMEM((_CHUNK,), jnp.int32),
      pltpu.VMEM((_CHUNK,), jnp.int32),
      pltpu.VMEM((_CHUNK,), jnp.int32),
  ] + [pltpu.SemaphoreType.DMA] * 8
  return pl.kernel(
      body,
      out_type=jax.ShapeDtypeStruct((_NC, N_pad, H), F32),
      mesh=mesh,
      scratch_types=scratch,
  ), N_pad


def kernel(x, edge_index, edge_attr, We, be, W1, b1, W2, b2):
  N, H = x.shape[0], We.shape[2]
  E, K = edge_attr.shape
  grain = _NW * _CHUNK * 2
  E_pad = (E + grain - 1) // grain * grain
  NCH = E_pad // (_NW * _CHUNK)
  n_extra = E_pad - E

  src = edge_index[0].astype(jnp.int32)
  dst = edge_index[1].astype(jnp.int32)
  if n_extra:
    src = jnp.concatenate([src, jnp.zeros((n_extra,), jnp.int32)])
    dst = jnp.concatenate([dst, jnp.full((n_extra,), N, jnp.int32)])
    edge_attr = jnp.concatenate(
        [edge_attr, jnp.zeros((n_extra, K), edge_attr.dtype)])

  sc_edge_pass, N_pad = _make_sc_edge_pass(N, E_pad, H, NCH)
  zeros = jnp.zeros((N_pad, H), dtype=F32)

  num_layers = We.shape[0]
  h = x
  for l in range(num_layers):
    e = _edge_proj(edge_attr, We[l], be[l])
    agg = sc_edge_pass(h, e, src, dst, zeros)
    h = _node_update(h, agg[0, :N], agg[1, :N], W1[l], b1[l], W2[l], b2[l],
                     relu_out=(l < num_layers - 1))
  return h
